# baseline (speedup 1.0000x reference)
import functools

import jax
import jax.numpy as jnp
import numpy as np
from jax import lax
from jax.experimental import pallas as pl
from jax.experimental.pallas import tpu as pltpu
from jax.experimental.pallas import tpu_sc as plsc

L, B, D = 200, 256, 64
K = 1000
N_ROWS = L * B

_SC_INFO = plsc.get_sparse_core_info()
_NC = _SC_INFO.num_cores
_NS = _SC_INFO.num_subcores
_NW = _NC * _NS
_CHUNK = N_ROWS // _NW
_PIECES = [(o, min(128, _CHUNK - o)) for o in range(0, _CHUNK, 128)]

_BL = 8
_NSTEPS = L // _BL


def _sc_gather_fn():
  mesh = plsc.VectorSubcoreMesh(core_axis_name="c", subcore_axis_name="s")

  @functools.partial(
      pl.kernel,
      mesh=mesh,
      compiler_params=pltpu.CompilerParams(use_tc_tiling_on_sc=False),
      out_type=(
          jax.ShapeDtypeStruct((N_ROWS, D), jnp.float32),
          jax.ShapeDtypeStruct((N_ROWS, D), jnp.float32),
      ),
      scratch_types=[
          pltpu.VMEM((_CHUNK,), jnp.int32),
          pltpu.VMEM((_CHUNK, D), jnp.float32),
          pltpu.SemaphoreType.DMA,
      ],
  )
  def sc_gather(node_idx_h, edge_idx_h, node_tab_h, edge_tab_h,
                node_out_h, edge_out_h, idx_v, rows_v, sem):
    wid = lax.axis_index("s") * _NC + lax.axis_index("c")
    base = wid * _CHUNK
    for tab_h, src_idx_h, out_h in (
        (node_tab_h, node_idx_h, node_out_h),
        (edge_tab_h, edge_idx_h, edge_out_h),
    ):
      pltpu.sync_copy(src_idx_h.at[pl.ds(base, _CHUNK)], idx_v)
      handles = []
      for off, sz in _PIECES:
        handles.append(
            pltpu.async_copy(
                tab_h.at[idx_v.at[pl.ds(off, sz)]],
                rows_v.at[pl.ds(off, sz)],
                sem,
            ))
      for h in handles:
        h.wait()
      pltpu.sync_copy(rows_v, out_h.at[pl.ds(base, _CHUNK)])

  return sc_gather


_sc_gather = _sc_gather_fn()


def _tc_body(nv_ref, ev_ref, vm_ref, vt_ref, pe_ref, out_ref):
  pe = pe_ref[...][:, None, :]
  x = vm_ref[...].reshape(_BL * B, K)
  y = jnp.dot(x, vt_ref[...], preferred_element_type=jnp.float32)
  out_ref[0] = nv_ref[...] + pe
  out_ref[1] = ev_ref[...] + pe
  out_ref[2] = y.reshape(_BL, B, D) + pe


_tc_combine = pl.pallas_call(
    _tc_body,
    grid=(_NSTEPS,),
    in_specs=[
        pl.BlockSpec((_BL, B, D), lambda l: (l, 0, 0)),
        pl.BlockSpec((_BL, B, D), lambda l: (l, 0, 0)),
        pl.BlockSpec((_BL, B, K), lambda l: (l, 0, 0)),
        pl.BlockSpec((K, D), lambda l: (0, 0)),
        pl.BlockSpec((_BL, D), lambda l: (l, 0)),
    ],
    out_specs=pl.BlockSpec((3, _BL, B, D), lambda l: (0, l, 0, 0)),
    out_shape=jax.ShapeDtypeStruct((3, L, B, D), jnp.float32),
)


def _pos_encoding_table():
  pos = jnp.arange(L, dtype=jnp.float32)[:, None]
  div = jnp.exp(jnp.arange(0, D, 2, dtype=jnp.float32) * (-np.log(10000.0) / D))
  pe = jnp.zeros((L, D), dtype=jnp.float32)
  pe = pe.at[:, 0::2].set(jnp.sin(pos * div))
  pe = pe.at[:, 1::2].set(jnp.cos(pos * div))
  return pe


def kernel(node_idx, edge_idx, node_val_mat, node_embed_table,
           edge_embed_table, val_tok_embed):
  pe = _pos_encoding_table()
  node_rows, edge_rows = _sc_gather(
      node_idx.reshape(-1), edge_idx.reshape(-1),
      node_embed_table, edge_embed_table)
  out = _tc_combine(
      node_rows.reshape(L, B, D), edge_rows.reshape(L, B, D),
      node_val_mat.reshape(L, B, K), val_tok_embed, pe)
  return out.reshape(3 * L, B, D)


# speedup vs baseline: 1.0184x; 1.0184x over previous
import functools

import jax
import jax.numpy as jnp
import numpy as np
from jax import lax
from jax.experimental import pallas as pl
from jax.experimental.pallas import tpu as pltpu
from jax.experimental.pallas import tpu_sc as plsc

L, B, D = 200, 256, 64
K = 1000
N_ROWS = L * B
DP = 128

_SC_INFO = plsc.get_sparse_core_info()
_NC = _SC_INFO.num_cores
_NS = _SC_INFO.num_subcores
_NW = _NC * _NS
_CHUNK = N_ROWS // _NW
_HALF = _CHUNK // 2
_PIECES = [(o, min(128, _HALF - o)) for o in range(0, _HALF, 128)]

_BL = 8
_NSTEPS = L // _BL


def _sc_gather_fn():
  mesh = plsc.VectorSubcoreMesh(core_axis_name="c", subcore_axis_name="s")

  @functools.partial(
      pl.kernel,
      mesh=mesh,
      out_type=(
          jax.ShapeDtypeStruct((N_ROWS, DP), jnp.float32),
          jax.ShapeDtypeStruct((N_ROWS, DP), jnp.float32),
      ),
      scratch_types=[
          pltpu.VMEM((_CHUNK,), jnp.int32),
          pltpu.VMEM((_HALF, DP), jnp.float32),
          pltpu.SemaphoreType.DMA,
      ],
  )
  def sc_gather(node_idx_h, edge_idx_h, node_tab_h, edge_tab_h,
                node_out_h, edge_out_h, idx_v, rows_v, sem):
    wid = lax.axis_index("s") * _NC + lax.axis_index("c")
    base = wid * _CHUNK
    for tab_h, src_idx_h, out_h in (
        (node_tab_h, node_idx_h, node_out_h),
        (edge_tab_h, edge_idx_h, edge_out_h),
    ):
      pltpu.sync_copy(src_idx_h.at[pl.ds(base, _CHUNK)], idx_v)
      for half in range(2):
        hoff = half * _HALF
        handles = []
        for off, sz in _PIECES:
          handles.append(
              pltpu.async_copy(
                  tab_h.at[idx_v.at[pl.ds(hoff + off, sz)]],
                  rows_v.at[pl.ds(off, sz)],
                  sem,
              ))
        for h in handles:
          h.wait()
        pltpu.sync_copy(rows_v, out_h.at[pl.ds(base + hoff, _HALF)])

  return sc_gather


_sc_gather = _sc_gather_fn()


def _tc_body(nv_ref, ev_ref, vm_ref, vt_ref, pe_ref, out_ref):
  pe = pe_ref[...][:, None, :]
  x = vm_ref[...].reshape(_BL * B, K)
  y = jnp.dot(x, vt_ref[...], preferred_element_type=jnp.float32)
  out_ref[0] = nv_ref[..., :D] + pe
  out_ref[1] = ev_ref[..., :D] + pe
  out_ref[2] = y.reshape(_BL, B, D) + pe


_tc_combine = pl.pallas_call(
    _tc_body,
    grid=(_NSTEPS,),
    in_specs=[
        pl.BlockSpec((_BL, B, DP), lambda l: (l, 0, 0)),
        pl.BlockSpec((_BL, B, DP), lambda l: (l, 0, 0)),
        pl.BlockSpec((_BL, B, K), lambda l: (l, 0, 0)),
        pl.BlockSpec((K, D), lambda l: (0, 0)),
        pl.BlockSpec((_BL, D), lambda l: (l, 0)),
    ],
    out_specs=pl.BlockSpec((3, _BL, B, D), lambda l: (0, l, 0, 0)),
    out_shape=jax.ShapeDtypeStruct((3, L, B, D), jnp.float32),
)


def _pos_encoding_table():
  pos = jnp.arange(L, dtype=jnp.float32)[:, None]
  div = jnp.exp(jnp.arange(0, D, 2, dtype=jnp.float32) * (-np.log(10000.0) / D))
  pe = jnp.zeros((L, D), dtype=jnp.float32)
  pe = pe.at[:, 0::2].set(jnp.sin(pos * div))
  pe = pe.at[:, 1::2].set(jnp.cos(pos * div))
  return pe


def kernel(node_idx, edge_idx, node_val_mat, node_embed_table,
           edge_embed_table, val_tok_embed):
  pe = _pos_encoding_table()
  node_tab_p = jnp.pad(node_embed_table, ((0, 0), (0, DP - D)))
  edge_tab_p = jnp.pad(edge_embed_table, ((0, 0), (0, DP - D)))
  node_rows, edge_rows = _sc_gather(
      node_idx.reshape(-1), edge_idx.reshape(-1), node_tab_p, edge_tab_p)
  out = _tc_combine(
      node_rows.reshape(L, B, DP), edge_rows.reshape(L, B, DP),
      node_val_mat.reshape(L, B, K), val_tok_embed, pe)
  return out.reshape(3 * L, B, D)


# speedup vs baseline: 1.0213x; 1.0028x over previous
import functools

import jax
import jax.numpy as jnp
import numpy as np
from jax import lax
from jax.experimental import pallas as pl
from jax.experimental.pallas import tpu as pltpu
from jax.experimental.pallas import tpu_sc as plsc

L, B, D = 200, 256, 64
K = 1000
N_ROWS = L * B
DP = 128

_SC_INFO = plsc.get_sparse_core_info()
_NC = _SC_INFO.num_cores
_NS = _SC_INFO.num_subcores
_NW = _NC * _NS
_CHUNK = N_ROWS // _NW
_HALF = _CHUNK // 2
_PIECES = [(o, min(128, _HALF - o)) for o in range(0, _HALF, 128)]

_BL = 8
_NSTEPS = L // _BL


def _sc_gather_fn():
  mesh = plsc.VectorSubcoreMesh(core_axis_name="c", subcore_axis_name="s")

  @functools.partial(
      pl.kernel,
      mesh=mesh,
      out_type=(
          jax.ShapeDtypeStruct((N_ROWS, DP), jnp.float32),
          jax.ShapeDtypeStruct((N_ROWS, DP), jnp.float32),
      ),
      scratch_types=[
          pltpu.VMEM((_CHUNK,), jnp.int32),
          pltpu.VMEM((_HALF, DP), jnp.float32),
          pltpu.SemaphoreType.DMA,
      ],
  )
  def sc_gather(node_idx_h, edge_idx_h, node_tab_h, edge_tab_h,
                node_out_h, edge_out_h, idx_v, rows_v, sem):
    wid = lax.axis_index("s") * _NC + lax.axis_index("c")
    base = wid * _CHUNK
    for tab_h, src_idx_h, out_h in (
        (node_tab_h, node_idx_h, node_out_h),
        (edge_tab_h, edge_idx_h, edge_out_h),
    ):
      pltpu.sync_copy(src_idx_h.at[pl.ds(base, _CHUNK)], idx_v)
      for half in range(2):
        hoff = half * _HALF
        handles = []
        for off, sz in _PIECES:
          handles.append(
              pltpu.async_copy(
                  tab_h.at[idx_v.at[pl.ds(hoff + off, sz)]],
                  rows_v.at[pl.ds(off, sz)],
                  sem,
              ))
        for h in handles:
          h.wait()
        pltpu.sync_copy(rows_v, out_h.at[pl.ds(base + hoff, _HALF)])

  return sc_gather


_sc_gather = _sc_gather_fn()


def _tc_body(nv_ref, ev_ref, vm_ref, vt_ref, pe_ref, out_ref):
  pe = pe_ref[...][:, None, :]
  x = vm_ref[...].reshape(_BL * B, K)
  y = jnp.dot(x, vt_ref[...], preferred_element_type=jnp.float32)
  out_ref[0] = nv_ref[..., :D] + pe
  out_ref[1] = ev_ref[..., :D] + pe
  out_ref[2] = y.reshape(_BL, B, D) + pe


_tc_combine = pl.pallas_call(
    _tc_body,
    grid=(_NSTEPS,),
    in_specs=[
        pl.BlockSpec((_BL, B, DP), lambda l: (l, 0, 0)),
        pl.BlockSpec((_BL, B, DP), lambda l: (l, 0, 0)),
        pl.BlockSpec((_BL, B, K), lambda l: (l, 0, 0)),
        pl.BlockSpec((K, D), lambda l: (0, 0)),
        pl.BlockSpec((_BL, D), lambda l: (l, 0)),
    ],
    out_specs=pl.BlockSpec((3, _BL, B, D), lambda l: (0, l, 0, 0)),
    out_shape=jax.ShapeDtypeStruct((3, L, B, D), jnp.float32),
)


def _pad_body(in_ref, out_ref):
  out_ref[:, :D] = in_ref[...]
  out_ref[:, D:] = jnp.zeros_like(out_ref[:, D:])


def _make_pad(n_rows, block_rows):
  return pl.pallas_call(
      _pad_body,
      grid=(n_rows // block_rows,),
      in_specs=[pl.BlockSpec((block_rows, D), lambda i: (i, 0))],
      out_specs=pl.BlockSpec((block_rows, DP), lambda i: (i, 0)),
      out_shape=jax.ShapeDtypeStruct((n_rows, DP), jnp.float32),
  )


_pad_node = _make_pad(100000, 2000)
_pad_edge = _make_pad(1000, 1000)


def _pos_encoding_table():
  pos = jnp.arange(L, dtype=jnp.float32)[:, None]
  div = jnp.exp(jnp.arange(0, D, 2, dtype=jnp.float32) * (-np.log(10000.0) / D))
  pe = jnp.zeros((L, D), dtype=jnp.float32)
  pe = pe.at[:, 0::2].set(jnp.sin(pos * div))
  pe = pe.at[:, 1::2].set(jnp.cos(pos * div))
  return pe


def kernel(node_idx, edge_idx, node_val_mat, node_embed_table,
           edge_embed_table, val_tok_embed):
  pe = _pos_encoding_table()
  node_tab_p = _pad_node(node_embed_table)
  edge_tab_p = _pad_edge(edge_embed_table)
  node_rows, edge_rows = _sc_gather(
      node_idx.reshape(-1), edge_idx.reshape(-1), node_tab_p, edge_tab_p)
  out = _tc_combine(
      node_rows.reshape(L, B, DP), edge_rows.reshape(L, B, DP),
      node_val_mat.reshape(L, B, K), val_tok_embed, pe)
  return out.reshape(3 * L, B, D)
